# baseline (speedup 1.0000x reference)
import dataclasses
import functools

import jax
import jax.numpy as jnp
from jax import lax
from jax.experimental import pallas as pl
from jax.experimental.pallas import tpu as pltpu
from jax.experimental.pallas import tpu_sc as plsc

_NC = 1
_NS = 16
_NW = _NC * _NS
_L = 16
_MAGIC = 12582912.0
_BIAS = 0x4B400000


@functools.partial(jax.jit, static_argnums=(2, 3))
def _sc_lookup(t, table, n, scale):
    chunk = n // _NW
    half = chunk // 2
    mesh = plsc.VectorSubcoreMesh(
        core_axis_name="c", subcore_axis_name="s", num_cores=_NC)
    cp = pltpu.CompilerParams()
    if "needs_layout_passes" in pltpu.CompilerParams.__dataclass_fields__:
        cp = dataclasses.replace(cp, needs_layout_passes=False)

    @functools.partial(
        pl.kernel,
        out_type=jax.ShapeDtypeStruct((n,), jnp.float32),
        mesh=mesh,
        compiler_params=cp,
        scratch_types=[
            pltpu.VMEM((table.shape[0],), jnp.float32),
            pltpu.VMEM((chunk,), jnp.float32),
            pltpu.SemaphoreType.DMA,
            pltpu.SemaphoreType.DMA,
        ],
    )
    def k(t_hbm, tbl_hbm, out_hbm, tbl_v, t_v, sem0, sem1):
        wid = lax.axis_index("s") * _NC + lax.axis_index("c")
        base = wid * chunk
        cp_t = pltpu.async_copy(t_hbm.at[pl.ds(base, chunk)], t_v, sem1)
        cp_tbl = pltpu.async_copy(tbl_hbm, tbl_v, sem0)
        cp_t.wait()
        cp_tbl.wait()

        quarter = chunk // 4
        outs = []
        for j in range(4):
            @plsc.parallel_loop(j * quarter, (j + 1) * quarter,
                                step=_L, unroll=8)
            def _(i):
                y = (t_v[pl.ds(i, _L)] * jnp.float32(scale)
                     + jnp.float32(_MAGIC))
                idx = plsc.bitcast(y, jnp.int32) - jnp.int32(_BIAS)
                t_v[pl.ds(i, _L)] = plsc.load_gather(tbl_v, [idx])

            outs.append(pltpu.async_copy(
                t_v.at[pl.ds(j * quarter, quarter)],
                out_hbm.at[pl.ds(base + j * quarter, quarter)],
                sem0 if j % 2 == 0 else sem1))
        for o in outs:
            o.wait()

    return k(t, table)


def kernel(t, type, alpha, sigma):
    T = alpha.shape[0] - 1
    table = alpha if type == 'alpha' else sigma
    return _sc_lookup(t, table, t.shape[0], float(T))


# speedup vs baseline: 1.0129x; 1.0129x over previous
import dataclasses
import functools

import jax
import jax.numpy as jnp
from jax import lax
from jax.experimental import pallas as pl
from jax.experimental.pallas import tpu as pltpu
from jax.experimental.pallas import tpu_sc as plsc

_NC = 1
_NS = 16
_NW = _NC * _NS
_L = 16
_MAGIC = 12582912.0
_BIAS = 0x4B400000


@functools.partial(jax.jit, static_argnums=(2, 3))
def _sc_lookup(t, table, n, scale):
    chunk = n // _NW
    half = chunk // 2
    mesh = plsc.VectorSubcoreMesh(
        core_axis_name="c", subcore_axis_name="s", num_cores=_NC)
    cp = pltpu.CompilerParams()
    if "needs_layout_passes" in pltpu.CompilerParams.__dataclass_fields__:
        cp = dataclasses.replace(cp, needs_layout_passes=False)

    @functools.partial(
        pl.kernel,
        out_type=jax.ShapeDtypeStruct((n,), jnp.float32),
        mesh=mesh,
        compiler_params=cp,
        scratch_types=[
            pltpu.VMEM((table.shape[0],), jnp.float32),
            pltpu.VMEM((chunk,), jnp.float32),
            pltpu.SemaphoreType.DMA,
            pltpu.SemaphoreType.DMA,
        ],
    )
    def k(t_hbm, tbl_hbm, out_hbm, tbl_v, t_v, sem0, sem1):
        wid = lax.axis_index("s") * _NC + lax.axis_index("c")
        base = wid * chunk
        cp_tbl = pltpu.async_copy(tbl_hbm, tbl_v, sem0)
        cp_t = pltpu.async_copy(t_hbm.at[pl.ds(base, chunk)], t_v, sem1)
        cp_t.wait()
        cp_tbl.wait()

        def gather_span(lo, hi):
            @plsc.parallel_loop(lo, hi, step=_L, unroll=8)
            def _(i):
                y = (t_v[pl.ds(i, _L)] * jnp.float32(scale)
                     + jnp.float32(_MAGIC))
                idx = plsc.bitcast(y, jnp.int32) - jnp.int32(_BIAS)
                t_v[pl.ds(i, _L)] = plsc.load_gather(tbl_v, [idx])

        gather_span(0, half)
        cp_o0 = pltpu.async_copy(
            t_v.at[pl.ds(0, half)], out_hbm.at[pl.ds(base, half)], sem0)
        gather_span(half, chunk)
        cp_o1 = pltpu.async_copy(
            t_v.at[pl.ds(half, half)],
            out_hbm.at[pl.ds(base + half, half)], sem1)
        cp_o0.wait()
        cp_o1.wait()

    return k(t, table)


def kernel(t, type, alpha, sigma):
    T = alpha.shape[0] - 1
    table = alpha if type == 'alpha' else sigma
    return _sc_lookup(t, table, t.shape[0], float(T))
